# Initial kernel scaffold; baseline (speedup 1.0000x reference)
import jax
import jax.numpy as jnp
from jax.experimental import pallas as pl


def kernel(sin_fan, cos, filt_w, filt_b, A_rows, A_cols, A_data):
    raise NotImplementedError("write your pallas kernel here")


# speedup vs baseline: 15.3210x; 15.3210x over previous
import functools

import jax
import jax.numpy as jnp
from jax import lax
from jax.experimental import pallas as pl
from jax.experimental.pallas import tpu as pltpu
from jax.experimental.pallas import tpu_sc as plsc

B = 8
H = 360
W = 357
SINSZ = H * W
NPIX = 65536
OUT = 256
KLEN = 713
NNZ = 2000000

NCORES = 2
NSUB = 16
NW = NCORES * NSUB
CB = 2000
NBLK = NNZ // CB
FULL_ROUNDS = NBLK // NW
LEFTOVER = NBLK - FULL_ROUNDS * NW


def _filter_body(sin_ref, cos_ref, m_ref, b_ref, out_ref):
    r = sin_ref[...] * cos_ref[...][None]
    x = r.reshape(B * H, W)
    y = lax.dot_general(x, m_ref[...], (((1,), (1,)), ((), ())),
                        preferred_element_type=jnp.float32)
    out_ref[...] = y + b_ref[0]


def _combine_body(p_ref, out_ref):
    x = p_ref[...]
    out_ref[...] = jnp.clip(x[0] + x[1], 0.0, 1.0)


def _spmm_sc(s_tab, rows, cols, data, zeros):
    mesh = plsc.VectorSubcoreMesh(core_axis_name="c", subcore_axis_name="s",
                                  num_cores=NCORES, num_subcores=NSUB)

    @functools.partial(
        pl.kernel,
        out_type=jax.ShapeDtypeStruct((NCORES * NPIX, B), jnp.float32),
        mesh=mesh,
        compiler_params=pltpu.CompilerParams(needs_layout_passes=False,
                                             use_tc_tiling_on_sc=False),
        scratch_types=[
            pltpu.VMEM_SHARED((NPIX, B), jnp.float32),
            pltpu.VMEM((CB,), jnp.int32),
            pltpu.VMEM((CB,), jnp.int32),
            pltpu.VMEM((CB,), jnp.float32),
            pltpu.VMEM((CB, B), jnp.float32),
            pltpu.VMEM((CB, B), jnp.float32),
        ],
    )
    def spmm_kernel(s_hbm, rows_hbm, cols_hbm, data_hbm, z_hbm, out_hbm,
                    acc, col_v, row_v, dat_v, gath_v, scl_v):
        cid = lax.axis_index("c")
        sid = lax.axis_index("s")
        wid = cid * NSUB + sid
        rows_per = NPIX // NSUB

        pltpu.sync_copy(z_hbm.at[pl.ds(sid * rows_per, rows_per)],
                        acc.at[pl.ds(sid * rows_per, rows_per)])
        plsc.subcore_barrier()

        io = lax.iota(jnp.int32, 16)
        hi = io // 8
        colg = io % 8

        def do_block(blk):
            base = blk * CB
            pltpu.sync_copy(cols_hbm.at[pl.ds(base, CB)], col_v)
            pltpu.sync_copy(rows_hbm.at[pl.ds(base, CB)], row_v)
            pltpu.sync_copy(data_hbm.at[pl.ds(base, CB)], dat_v)
            pltpu.sync_copy(s_hbm.at[col_v], gath_v)

            @pl.loop(0, CB // 2)
            def _(g):
                rowg = g * 2 + hi
                dv = plsc.load_gather(dat_v, [rowg])
                sv = plsc.load_gather(gath_v, [rowg, colg])
                plsc.store_scatter(scl_v, [rowg, colg], sv * dv)

            pltpu.sync_copy(scl_v, acc.at[row_v], add=True)

        @pl.loop(0, FULL_ROUNDS)
        def _(r):
            do_block(r * NW + wid)

        @pl.when(wid < LEFTOVER)
        def _():
            do_block(FULL_ROUNDS * NW + wid)

        plsc.subcore_barrier()
        pltpu.sync_copy(
            acc.at[pl.ds(sid * rows_per, rows_per)],
            out_hbm.at[pl.ds(cid * NPIX + sid * rows_per, rows_per)])

    return spmm_kernel(s_tab, rows, cols, data, zeros)


def kernel(sin_fan, cos, filt_w, filt_b, A_rows, A_cols, A_data):
    i = jnp.arange(W)
    m = filt_w[(KLEN - 1) // 2 + i[None, :] - i[:, None]]

    y = pl.pallas_call(
        _filter_body,
        out_shape=jax.ShapeDtypeStruct((B * H, W), jnp.float32),
    )(sin_fan, cos, m, filt_b)

    s_tab = jnp.transpose(y.reshape(B, H, W), (1, 2, 0)).reshape(SINSZ, B)

    zeros = jnp.zeros((NPIX, B), jnp.float32)
    parts = _spmm_sc(s_tab, A_rows, A_cols, A_data, zeros)

    fbp = pl.pallas_call(
        _combine_body,
        out_shape=jax.ShapeDtypeStruct((NPIX * B,), jnp.float32),
    )(parts.reshape(NCORES, NPIX * B))

    return fbp.reshape(NPIX, B).T.reshape(B, OUT, OUT, 1)


# speedup vs baseline: 43.3289x; 2.8281x over previous
import functools

import jax
import jax.numpy as jnp
from jax import lax
from jax.experimental import pallas as pl
from jax.experimental.pallas import tpu as pltpu
from jax.experimental.pallas import tpu_sc as plsc

B = 8
H = 360
W = 357
SINSZ = H * W
NPIX = 65536
OUT = 256
KLEN = 713
NNZ = 2000000

NCORES = 2
NSUB = 16
NW = NCORES * NSUB
CB = 2000
NBLK = NNZ // CB
FULL_ROUNDS = NBLK // NW
LEFTOVER = NBLK - FULL_ROUNDS * NW


def _filter_body(sin_ref, cos_ref, m_ref, b_ref, out_ref):
    r = sin_ref[...] * cos_ref[...][None]
    x = r.reshape(B * H, W)
    y = lax.dot_general(x, m_ref[...], (((1,), (1,)), ((), ())),
                        preferred_element_type=jnp.float32)
    out_ref[...] = y + b_ref[0]


def _combine_body(p_ref, out_ref):
    x = p_ref[...]
    out_ref[...] = jnp.clip(x[0] + x[1], 0.0, 1.0)


def _spmm_sc(s_tab, rows, cols, data, zeros):
    mesh = plsc.VectorSubcoreMesh(core_axis_name="c", subcore_axis_name="s",
                                  num_cores=NCORES, num_subcores=NSUB)

    @functools.partial(
        pl.kernel,
        out_type=jax.ShapeDtypeStruct((NCORES * NPIX, B), jnp.float32),
        mesh=mesh,
        compiler_params=pltpu.CompilerParams(needs_layout_passes=False,
                                             use_tc_tiling_on_sc=False),
        scratch_types=[
            pltpu.VMEM_SHARED((NPIX, B), jnp.float32),
            pltpu.VMEM((CB,), jnp.int32),
            pltpu.VMEM((CB,), jnp.int32),
            pltpu.VMEM((CB,), jnp.float32),
            pltpu.VMEM((CB, B), jnp.float32),
            pltpu.VMEM((CB, B), jnp.float32),
        ],
    )
    def spmm_kernel(s_hbm, rows_hbm, cols_hbm, data_hbm, z_hbm, out_hbm,
                    acc, col_v, row_v, dat_v, gath_v, scl_v):
        cid = lax.axis_index("c")
        sid = lax.axis_index("s")
        wid = cid * NSUB + sid
        rows_per = NPIX // NSUB

        pltpu.sync_copy(z_hbm.at[pl.ds(sid * rows_per, rows_per)],
                        acc.at[pl.ds(sid * rows_per, rows_per)])
        plsc.subcore_barrier()

        io = lax.iota(jnp.int32, 16)
        hi = io // 8
        colg = io % 8

        def do_block(blk):
            base = blk * CB
            pltpu.sync_copy(cols_hbm.at[pl.ds(base, CB)], col_v)
            pltpu.sync_copy(rows_hbm.at[pl.ds(base, CB)], row_v)
            pltpu.sync_copy(data_hbm.at[pl.ds(base, CB)], dat_v)
            pltpu.sync_copy(s_hbm.at[col_v], gath_v)

            @functools.partial(plsc.parallel_loop, 0, CB // 2, unroll=8)
            def _(g):
                rowg = g * 2 + hi
                dv = plsc.load_gather(dat_v, [rowg])
                sv = plsc.load_gather(gath_v, [rowg, colg])
                plsc.store_scatter(scl_v, [rowg, colg], sv * dv)

            pltpu.sync_copy(scl_v, acc.at[row_v], add=True)

        @pl.loop(0, FULL_ROUNDS)
        def _(r):
            do_block(r * NW + wid)

        @pl.when(wid < LEFTOVER)
        def _():
            do_block(FULL_ROUNDS * NW + wid)

        plsc.subcore_barrier()
        pltpu.sync_copy(
            acc.at[pl.ds(sid * rows_per, rows_per)],
            out_hbm.at[pl.ds(cid * NPIX + sid * rows_per, rows_per)])

    return spmm_kernel(s_tab, rows, cols, data, zeros)


def kernel(sin_fan, cos, filt_w, filt_b, A_rows, A_cols, A_data):
    vp = jnp.concatenate([filt_w, jnp.zeros((1,), jnp.float32)])
    flat = jnp.broadcast_to(vp, (W, KLEN + 1)).reshape(-1)
    m = flat[: W * KLEN].reshape(W, KLEN)[:, (KLEN - 1) // 2:]

    y = pl.pallas_call(
        _filter_body,
        out_shape=jax.ShapeDtypeStruct((B * H, W), jnp.float32),
    )(sin_fan, cos, m, filt_b)

    s_tab = jnp.transpose(y.reshape(B, H, W), (1, 2, 0)).reshape(SINSZ, B)

    zeros = jnp.zeros((NPIX, B), jnp.float32)
    parts = _spmm_sc(s_tab, A_rows, A_cols, A_data, zeros)

    fbp = pl.pallas_call(
        _combine_body,
        out_shape=jax.ShapeDtypeStruct((NPIX * B,), jnp.float32),
    )(parts.reshape(NCORES, NPIX * B))

    return fbp.reshape(NPIX, B).T.reshape(B, OUT, OUT, 1)
